# baseline (speedup 1.0000x reference)
import functools

import jax
import jax.numpy as jnp
from jax import lax
from jax.experimental import pallas as pl
from jax.experimental.pallas import tpu as pltpu


def _theta_kernel(theta_ref, tw0, tb0, tw1, tb1, tw2, tb2,
                  sw2, sb2, wl0, bl0, tb_out, ws_out):
    t = theta_ref[...]
    t = jnp.maximum(jnp.dot(t, tw0[...], preferred_element_type=jnp.float32)
                    + tb0[...], 0.0)
    t = jnp.maximum(jnp.dot(t, tw1[...], preferred_element_type=jnp.float32)
                    + tb1[...], 0.0)
    wl0_t = wl0[0:32, :]
    wl0_s = wl0[32:64, :]
    w_t = jnp.dot(tw2[...], wl0_t, preferred_element_type=jnp.float32)
    b_t = (jnp.dot(tb2[...], wl0_t, preferred_element_type=jnp.float32)
           + jnp.dot(sb2[...], wl0_s, preferred_element_type=jnp.float32)
           + bl0[...])
    tb_out[...] = (jnp.dot(t, w_t, preferred_element_type=jnp.float32) + b_t)
    ws_out[...] = jnp.dot(sw2[...], wl0_s, preferred_element_type=jnp.float32)


def _sim_kernel(x_ref, tb_ref, w0, b0, w1, b1, ws,
                lw1, lb1, lw2t, lb2, o_ref):
    x = x_ref[0].astype(jnp.bfloat16)
    h = jnp.dot(x, w0[...].astype(jnp.bfloat16),
                preferred_element_type=jnp.float32).astype(jnp.bfloat16)
    h = jnp.maximum(h + b0[...].astype(jnp.bfloat16), 0.0)
    h = jnp.dot(h, w1[...].astype(jnp.bfloat16),
                preferred_element_type=jnp.float32).astype(jnp.bfloat16)
    h = jnp.maximum(h + b1[...].astype(jnp.bfloat16), 0.0)
    h = jnp.dot(h, ws[...].astype(jnp.bfloat16),
                preferred_element_type=jnp.float32).astype(jnp.bfloat16)
    h = jnp.maximum(h + tb_ref[0].astype(jnp.bfloat16), 0.0)
    h = jnp.dot(h, lw1[...].astype(jnp.bfloat16),
                preferred_element_type=jnp.float32).astype(jnp.bfloat16)
    h = jnp.maximum(h + lb1[...].astype(jnp.bfloat16), 0.0)
    row = lax.dot_general(lw2t[...].astype(jnp.bfloat16), h,
                          (((1,), (1,)), ((), ())),
                          preferred_element_type=jnp.float32) + lb2[...]
    o_ref[0] = row


def _rep(arr):
    zeros = (0,) * arr.ndim
    return pl.BlockSpec(arr.shape, lambda *_: zeros)


def kernel(theta, x_target, te_w0, te_b0, te_w1, te_b1, te_w2, te_b2,
           se_w0, se_b0, se_w1, se_b1, se_w2, se_b2,
           lm_w0, lm_b0, lm_w1, lm_b1, lm_w2, lm_b2):
    B, theta_dim = theta.shape
    _, N, sim_dim = x_target.shape
    h0 = lm_w0.shape[1]

    tb, ws = pl.pallas_call(
        _theta_kernel,
        out_shape=[
            jax.ShapeDtypeStruct((B, h0), jnp.float32),
            jax.ShapeDtypeStruct((se_w2.shape[0], h0), jnp.float32),
        ],
        compiler_params=pltpu.CompilerParams(
            vmem_limit_bytes=64 * 1024 * 1024,
        ),
    )(theta, te_w0, te_b0, te_w1, te_b1, te_w2, te_b2,
      se_w2, se_b2, lm_w0, lm_b0)

    tn = 512
    while N % tn:
        tn //= 2
    lm_w2t = lm_w2.reshape(1, lm_w2.shape[0])
    tb3 = tb.reshape(B, 1, h0)

    in_specs = [
        pl.BlockSpec((1, tn, sim_dim), lambda b, n: (b, n, 0)),
        pl.BlockSpec((1, 1, h0), lambda b, n: (b, 0, 0)),
        _rep(se_w0), _rep(se_b0), _rep(se_w1), _rep(se_b1),
        _rep(ws), _rep(lm_w1), _rep(lm_b1),
        _rep(lm_w2t), _rep(lm_b2),
    ]
    out = pl.pallas_call(
        _sim_kernel,
        out_shape=jax.ShapeDtypeStruct((B, 1, N), jnp.float32),
        grid_spec=pltpu.PrefetchScalarGridSpec(
            num_scalar_prefetch=0,
            grid=(B, N // tn),
            in_specs=in_specs,
            out_specs=pl.BlockSpec((1, 1, tn), lambda b, n: (b, 0, n)),
        ),
        compiler_params=pltpu.CompilerParams(
            dimension_semantics=("parallel", "parallel"),
            vmem_limit_bytes=64 * 1024 * 1024,
        ),
    )(x_target, tb3, se_w0, se_b0, se_w1, se_b1, ws,
      lm_w1, lm_b1, lm_w2t, lm_b2)

    return out.reshape(B, N, 1)


# speedup vs baseline: 4.8381x; 4.8381x over previous
import functools

import jax
import jax.numpy as jnp
from jax import lax
from jax.experimental import pallas as pl
from jax.experimental.pallas import tpu as pltpu


def _theta_kernel(theta_ref, tw0, tb0, tw1, tb1, tw2, tb2,
                  sw2, sb2, wl0, bl0, tb_out, ws_out):
    t = theta_ref[...]
    t = jnp.maximum(jnp.dot(t, tw0[...], preferred_element_type=jnp.float32)
                    + tb0[...], 0.0)
    t = jnp.maximum(jnp.dot(t, tw1[...], preferred_element_type=jnp.float32)
                    + tb1[...], 0.0)
    wl0_t = wl0[0:32, :]
    wl0_s = wl0[32:64, :]
    w_t = jnp.dot(tw2[...], wl0_t, preferred_element_type=jnp.float32)
    b_t = (jnp.dot(tb2[...], wl0_t, preferred_element_type=jnp.float32)
           + jnp.dot(sb2[...], wl0_s, preferred_element_type=jnp.float32)
           + bl0[...])
    tb_out[...] = (jnp.dot(t, w_t, preferred_element_type=jnp.float32) + b_t)
    ws_out[...] = jnp.dot(sw2[...], wl0_s, preferred_element_type=jnp.float32)


def _sim_kernel(nb, n_tgt, x_ref, tb_ref, w0, b0, w1, b1, ws,
                lw1, lb1, lw2t, lb2, o_ref):
    w0b = w0[...].astype(jnp.bfloat16)
    parts = []
    for i in range(nb):
        xi = x_ref[i].astype(jnp.bfloat16)
        parts.append(lax.dot_general(xi, w0b, (((0,), (0,)), ((), ())),
                                     preferred_element_type=jnp.float32))
    h = jnp.concatenate(parts, axis=0).astype(jnp.bfloat16)
    h = jnp.maximum(h + b0[...].astype(jnp.bfloat16), 0.0)
    h = jnp.dot(h, w1[...].astype(jnp.bfloat16),
                preferred_element_type=jnp.float32).astype(jnp.bfloat16)
    h = jnp.maximum(h + b1[...].astype(jnp.bfloat16), 0.0)
    h = jnp.dot(h, ws[...].astype(jnp.bfloat16),
                preferred_element_type=jnp.float32).astype(jnp.bfloat16)
    h = h.reshape(nb, n_tgt, h.shape[-1]) + tb_ref[...].astype(jnp.bfloat16)
    h = jnp.maximum(h, 0.0).reshape(nb * n_tgt, h.shape[-1])
    h = jnp.dot(h, lw1[...].astype(jnp.bfloat16),
                preferred_element_type=jnp.float32).astype(jnp.bfloat16)
    h = jnp.maximum(h + lb1[...].astype(jnp.bfloat16), 0.0)
    row = lax.dot_general(lw2t[...].astype(jnp.bfloat16), h,
                          (((1,), (1,)), ((), ())),
                          preferred_element_type=jnp.float32) + lb2[...]
    o_ref[...] = row


def _rep(arr):
    zeros = (0,) * arr.ndim
    return pl.BlockSpec(arr.shape, lambda *_: zeros)


def kernel(theta, x_target, te_w0, te_b0, te_w1, te_b1, te_w2, te_b2,
           se_w0, se_b0, se_w1, se_b1, se_w2, se_b2,
           lm_w0, lm_b0, lm_w1, lm_b1, lm_w2, lm_b2):
    B, theta_dim = theta.shape
    _, N, sim_dim = x_target.shape
    h0 = lm_w0.shape[1]

    tb, ws = pl.pallas_call(
        _theta_kernel,
        out_shape=[
            jax.ShapeDtypeStruct((B, h0), jnp.float32),
            jax.ShapeDtypeStruct((se_w2.shape[0], h0), jnp.float32),
        ],
        compiler_params=pltpu.CompilerParams(
            vmem_limit_bytes=64 * 1024 * 1024,
        ),
    )(theta, te_w0, te_b0, te_w1, te_b1, te_w2, te_b2,
      se_w2, se_b2, lm_w0, lm_b0)

    nb = 4
    while B % nb:
        nb //= 2
    grid = (B // nb,)
    xt = jnp.swapaxes(x_target, 1, 2)
    lm_w2t = lm_w2.reshape(1, lm_w2.shape[0])
    tb3 = tb.reshape(B, 1, h0)

    in_specs = [
        pl.BlockSpec((nb, sim_dim, N), lambda i: (i, 0, 0)),
        pl.BlockSpec((nb, 1, h0), lambda i: (i, 0, 0)),
        _rep(se_w0), _rep(se_b0), _rep(se_w1), _rep(se_b1),
        _rep(ws), _rep(lm_w1), _rep(lm_b1),
        _rep(lm_w2t), _rep(lm_b2),
    ]
    out = pl.pallas_call(
        functools.partial(_sim_kernel, nb, N),
        out_shape=jax.ShapeDtypeStruct((1, B * N), jnp.float32),
        grid=grid,
        in_specs=in_specs,
        out_specs=pl.BlockSpec((1, nb * N), lambda i: (0, i)),
        compiler_params=pltpu.CompilerParams(
            dimension_semantics=("parallel",),
            vmem_limit_bytes=64 * 1024 * 1024,
        ),
    )(xt, tb3, se_w0, se_b0, se_w1, se_b1, ws,
      lm_w1, lm_b1, lm_w2t, lm_b2)

    return out.reshape(B, N, 1)


# speedup vs baseline: 5.0938x; 1.0529x over previous
import functools

import jax
import jax.numpy as jnp
from jax import lax
from jax.experimental import pallas as pl
from jax.experimental.pallas import tpu as pltpu


def _theta_kernel(theta_ref, tw0, tb0, tw1, tb1, tw2, tb2,
                  sw2, sb2, wl0, bl0, tb_out, ws_out):
    t = theta_ref[...]
    t = jnp.maximum(jnp.dot(t, tw0[...], preferred_element_type=jnp.float32)
                    + tb0[...], 0.0)
    t = jnp.maximum(jnp.dot(t, tw1[...], preferred_element_type=jnp.float32)
                    + tb1[...], 0.0)
    wl0_t = wl0[0:32, :]
    wl0_s = wl0[32:64, :]
    w_t = jnp.dot(tw2[...], wl0_t, preferred_element_type=jnp.float32)
    b_t = (jnp.dot(tb2[...], wl0_t, preferred_element_type=jnp.float32)
           + jnp.dot(sb2[...], wl0_s, preferred_element_type=jnp.float32)
           + bl0[...])
    tb_out[...] = (jnp.dot(t, w_t, preferred_element_type=jnp.float32) + b_t)
    ws_out[...] = jnp.dot(sw2[...], wl0_s, preferred_element_type=jnp.float32)


def _sim_kernel(nb, n_tgt, x_ref, tb_ref, w0, b0, w1, b1, ws,
                lw1, lb1, lw2t, lb2, o_ref):
    w0b = w0[...].astype(jnp.bfloat16)
    parts = []
    for i in range(nb):
        xi = x_ref[i].astype(jnp.bfloat16)
        parts.append(lax.dot_general(xi, w0b, (((0,), (0,)), ((), ())),
                                     preferred_element_type=jnp.float32))
    h = jnp.concatenate(parts, axis=0).astype(jnp.bfloat16)
    h = jnp.maximum(h + b0[...].astype(jnp.bfloat16), 0.0)
    h = jnp.dot(h, w1[...].astype(jnp.bfloat16),
                preferred_element_type=jnp.float32).astype(jnp.bfloat16)
    h = jnp.maximum(h + b1[...].astype(jnp.bfloat16), 0.0)
    h = jnp.dot(h, ws[...].astype(jnp.bfloat16),
                preferred_element_type=jnp.float32).astype(jnp.bfloat16)
    h = h.reshape(nb, n_tgt, h.shape[-1]) + tb_ref[...].astype(jnp.bfloat16)
    h = jnp.maximum(h, 0.0).reshape(nb * n_tgt, h.shape[-1])
    h = jnp.dot(h, lw1[...].astype(jnp.bfloat16),
                preferred_element_type=jnp.float32).astype(jnp.bfloat16)
    h = jnp.maximum(h + lb1[...].astype(jnp.bfloat16), 0.0)
    row = lax.dot_general(lw2t[...].astype(jnp.bfloat16), h,
                          (((1,), (1,)), ((), ())),
                          preferred_element_type=jnp.float32) + lb2[...]
    o_ref[...] = row


def _rep(arr):
    zeros = (0,) * arr.ndim
    return pl.BlockSpec(arr.shape, lambda *_: zeros)


def kernel(theta, x_target, te_w0, te_b0, te_w1, te_b1, te_w2, te_b2,
           se_w0, se_b0, se_w1, se_b1, se_w2, se_b2,
           lm_w0, lm_b0, lm_w1, lm_b1, lm_w2, lm_b2):
    B, theta_dim = theta.shape
    _, N, sim_dim = x_target.shape
    h0 = lm_w0.shape[1]

    tb, ws = pl.pallas_call(
        _theta_kernel,
        out_shape=[
            jax.ShapeDtypeStruct((B, h0), jnp.float32),
            jax.ShapeDtypeStruct((se_w2.shape[0], h0), jnp.float32),
        ],
        compiler_params=pltpu.CompilerParams(
            vmem_limit_bytes=64 * 1024 * 1024,
        ),
    )(theta, te_w0, te_b0, te_w1, te_b1, te_w2, te_b2,
      se_w2, se_b2, lm_w0, lm_b0)

    nb = 8
    while B % nb:
        nb //= 2
    xt = jnp.swapaxes(x_target, 1, 2)
    lm_w2t = lm_w2.reshape(1, lm_w2.shape[0])
    tb3 = tb.reshape(B, 1, h0)

    in_specs = [
        pl.BlockSpec((nb, sim_dim, N), lambda i: (i, 0, 0)),
        pl.BlockSpec((nb, 1, h0), lambda i: (i, 0, 0)),
        _rep(se_w0), _rep(se_b0), _rep(se_w1), _rep(se_b1),
        _rep(ws), _rep(lm_w1), _rep(lm_b1),
        _rep(lm_w2t), _rep(lm_b2),
    ]
    out = pl.pallas_call(
        functools.partial(_sim_kernel, nb, N),
        out_shape=jax.ShapeDtypeStruct((1, B * N), jnp.float32),
        grid=(B // nb,),
        in_specs=in_specs,
        out_specs=pl.BlockSpec((1, nb * N), lambda i: (0, i)),
        compiler_params=pltpu.CompilerParams(
            dimension_semantics=("parallel",),
            vmem_limit_bytes=64 * 1024 * 1024,
        ),
    )(xt, tb3, se_w0, se_b0, se_w1, se_b1, ws,
      lm_w1, lm_b1, lm_w2t, lm_b2)

    return out.reshape(B, N, 1)


# speedup vs baseline: 5.2220x; 1.0252x over previous
import functools

import jax
import jax.numpy as jnp
from jax import lax
from jax.experimental import pallas as pl
from jax.experimental.pallas import tpu as pltpu


def _theta_kernel(theta_ref, tw0, tb0, tw1, tb1, tw2, tb2,
                  sw2, sb2, wl0, bl0, tb_out, ws_out):
    t = theta_ref[...]
    t = jnp.maximum(jnp.dot(t, tw0[...], preferred_element_type=jnp.float32)
                    + tb0[...], 0.0)
    t = jnp.maximum(jnp.dot(t, tw1[...], preferred_element_type=jnp.float32)
                    + tb1[...], 0.0)
    wl0_t = wl0[0:32, :]
    wl0_s = wl0[32:64, :]
    w_t = jnp.dot(tw2[...], wl0_t, preferred_element_type=jnp.float32)
    b_t = (jnp.dot(tb2[...], wl0_t, preferred_element_type=jnp.float32)
           + jnp.dot(sb2[...], wl0_s, preferred_element_type=jnp.float32)
           + bl0[...])
    tb_out[...] = (jnp.dot(t, w_t, preferred_element_type=jnp.float32) + b_t)
    ws_out[...] = jnp.dot(sw2[...], wl0_s, preferred_element_type=jnp.float32)


def _sim_kernel(nb, n_tgt, x_ref, tb_ref, w0, b0, w1, b1, ws,
                lw1, lb1, lw2t, lb2, o_ref):
    w0b = w0[...].astype(jnp.bfloat16)
    parts = []
    for i in range(nb):
        xi = x_ref[i].astype(jnp.bfloat16)
        parts.append(lax.dot_general(xi, w0b, (((0,), (0,)), ((), ())),
                                     preferred_element_type=jnp.float32))
    h = jnp.concatenate(parts, axis=0).astype(jnp.bfloat16)
    h = jnp.maximum(h + b0[...].astype(jnp.bfloat16), 0.0)
    h = jnp.dot(h, w1[...].astype(jnp.bfloat16),
                preferred_element_type=jnp.float32).astype(jnp.bfloat16)
    h = jnp.maximum(h + b1[...].astype(jnp.bfloat16), 0.0)
    h = jnp.dot(h, ws[...].astype(jnp.bfloat16),
                preferred_element_type=jnp.float32).astype(jnp.bfloat16)
    h = h.reshape(nb, n_tgt, h.shape[-1]) + tb_ref[...].astype(jnp.bfloat16)
    h = jnp.maximum(h, 0.0).reshape(nb * n_tgt, h.shape[-1])
    h = jnp.dot(h, lw1[...].astype(jnp.bfloat16),
                preferred_element_type=jnp.float32).astype(jnp.bfloat16)
    h = jnp.maximum(h + lb1[...].astype(jnp.bfloat16), 0.0)
    row = lax.dot_general(lw2t[...].astype(jnp.bfloat16), h,
                          (((1,), (1,)), ((), ())),
                          preferred_element_type=jnp.float32) + lb2[...]
    o_ref[...] = row


def _rep(arr):
    zeros = (0,) * arr.ndim
    return pl.BlockSpec(arr.shape, lambda *_: zeros)


def kernel(theta, x_target, te_w0, te_b0, te_w1, te_b1, te_w2, te_b2,
           se_w0, se_b0, se_w1, se_b1, se_w2, se_b2,
           lm_w0, lm_b0, lm_w1, lm_b1, lm_w2, lm_b2):
    B, theta_dim = theta.shape
    _, N, sim_dim = x_target.shape
    h0 = lm_w0.shape[1]

    tb, ws = pl.pallas_call(
        _theta_kernel,
        out_shape=[
            jax.ShapeDtypeStruct((B, h0), jnp.float32),
            jax.ShapeDtypeStruct((se_w2.shape[0], h0), jnp.float32),
        ],
        compiler_params=pltpu.CompilerParams(
            vmem_limit_bytes=64 * 1024 * 1024,
        ),
    )(theta, te_w0, te_b0, te_w1, te_b1, te_w2, te_b2,
      se_w2, se_b2, lm_w0, lm_b0)

    nb = 16
    while B % nb:
        nb //= 2
    xt = jnp.swapaxes(x_target, 1, 2)
    lm_w2t = lm_w2.reshape(1, lm_w2.shape[0])
    tb3 = tb.reshape(B, 1, h0)

    in_specs = [
        pl.BlockSpec((nb, sim_dim, N), lambda i: (i, 0, 0)),
        pl.BlockSpec((nb, 1, h0), lambda i: (i, 0, 0)),
        _rep(se_w0), _rep(se_b0), _rep(se_w1), _rep(se_b1),
        _rep(ws), _rep(lm_w1), _rep(lm_b1),
        _rep(lm_w2t), _rep(lm_b2),
    ]
    out = pl.pallas_call(
        functools.partial(_sim_kernel, nb, N),
        out_shape=jax.ShapeDtypeStruct((1, B * N), jnp.float32),
        grid=(B // nb,),
        in_specs=in_specs,
        out_specs=pl.BlockSpec((1, nb * N), lambda i: (0, i)),
        compiler_params=pltpu.CompilerParams(
            dimension_semantics=("parallel",),
            vmem_limit_bytes=64 * 1024 * 1024,
        ),
    )(xt, tb3, se_w0, se_b0, se_w1, se_b1, ws,
      lm_w1, lm_b1, lm_w2t, lm_b2)

    return out.reshape(B, N, 1)


# speedup vs baseline: 5.3949x; 1.0331x over previous
import functools

import jax
import jax.numpy as jnp
from jax import lax
from jax.experimental import pallas as pl
from jax.experimental.pallas import tpu as pltpu


def _theta_kernel(theta_t_ref, tw0, tb0, tw1, tb1, tw2_t, tb2,
                  sw2_t, sb2, wl0, bl0, tb_out, ws_out):
    ta = (((0,), (0,)), ((), ()))
    t = lax.dot_general(theta_t_ref[...], tw0[...], ta,
                        preferred_element_type=jnp.float32)
    t = jnp.maximum(t + tb0[...], 0.0)
    t = jnp.maximum(jnp.dot(t, tw1[...], preferred_element_type=jnp.float32)
                    + tb1[...], 0.0)
    wl0_t = wl0[0:32, :]
    wl0_s = wl0[32:64, :]
    w_t = lax.dot_general(tw2_t[...], wl0_t, ta,
                          preferred_element_type=jnp.float32)
    b_t = (jnp.dot(tb2[...], wl0_t, preferred_element_type=jnp.float32)
           + jnp.dot(sb2[...], wl0_s, preferred_element_type=jnp.float32)
           + bl0[...])
    tb_out[...] = (jnp.dot(t, w_t, preferred_element_type=jnp.float32) + b_t)
    ws_out[...] = lax.dot_general(sw2_t[...], wl0_s, ta,
                                  preferred_element_type=jnp.float32)


def _sim_kernel(nb, n_tgt, x_ref, tb_ref, w0, b0, w1, b1, ws,
                lw1_t, lb1, lw2t, lb2, o_ref):
    w0b = w0[...].astype(jnp.bfloat16)
    parts = []
    for i in range(nb):
        xi = x_ref[i].astype(jnp.bfloat16)
        parts.append(lax.dot_general(xi, w0b, (((0,), (0,)), ((), ())),
                                     preferred_element_type=jnp.float32))
    h = jnp.concatenate(parts, axis=0).astype(jnp.bfloat16)
    h = jnp.maximum(h + b0[...].astype(jnp.bfloat16), 0.0)
    h = jnp.dot(h, w1[...].astype(jnp.bfloat16),
                preferred_element_type=jnp.float32).astype(jnp.bfloat16)
    h = jnp.maximum(h + b1[...].astype(jnp.bfloat16), 0.0)
    h = jnp.dot(h, ws[...].astype(jnp.bfloat16),
                preferred_element_type=jnp.float32).astype(jnp.bfloat16)
    h = h.reshape(nb, n_tgt, h.shape[-1]) + tb_ref[...].astype(jnp.bfloat16)
    h = jnp.maximum(h, 0.0).reshape(nb * n_tgt, h.shape[-1])
    h = lax.dot_general(h, lw1_t[...].astype(jnp.bfloat16),
                        (((1,), (1,)), ((), ())),
                        preferred_element_type=jnp.float32).astype(jnp.bfloat16)
    h = jnp.maximum(h + lb1[...].astype(jnp.bfloat16), 0.0)
    row = lax.dot_general(lw2t[...].astype(jnp.bfloat16), h,
                          (((1,), (1,)), ((), ())),
                          preferred_element_type=jnp.float32) + lb2[...]
    o_ref[...] = row


def _rep(arr):
    zeros = (0,) * arr.ndim
    return pl.BlockSpec(arr.shape, lambda *_: zeros)


def kernel(theta, x_target, te_w0, te_b0, te_w1, te_b1, te_w2, te_b2,
           se_w0, se_b0, se_w1, se_b1, se_w2, se_b2,
           lm_w0, lm_b0, lm_w1, lm_b1, lm_w2, lm_b2):
    B, theta_dim = theta.shape
    _, N, sim_dim = x_target.shape
    h0 = lm_w0.shape[1]

    tb, ws = pl.pallas_call(
        _theta_kernel,
        out_shape=[
            jax.ShapeDtypeStruct((B, h0), jnp.float32),
            jax.ShapeDtypeStruct((se_w2.shape[0], h0), jnp.float32),
        ],
        compiler_params=pltpu.CompilerParams(
            vmem_limit_bytes=64 * 1024 * 1024,
        ),
    )(jnp.swapaxes(theta, 0, 1), te_w0, te_b0, te_w1, te_b1,
      jnp.swapaxes(te_w2, 0, 1), te_b2,
      jnp.swapaxes(se_w2, 0, 1), se_b2, lm_w0, lm_b0)

    nb = 16
    while B % nb:
        nb //= 2
    xt = jnp.swapaxes(x_target, 1, 2)
    lw1t = jnp.swapaxes(lm_w1, 0, 1)
    lm_w2t = lm_w2.reshape(1, lm_w2.shape[0])
    tb3 = tb.reshape(B, 1, h0)

    in_specs = [
        pl.BlockSpec((nb, sim_dim, N), lambda i: (i, 0, 0)),
        pl.BlockSpec((nb, 1, h0), lambda i: (i, 0, 0)),
        _rep(se_w0), _rep(se_b0), _rep(se_w1), _rep(se_b1),
        _rep(ws), _rep(lw1t), _rep(lm_b1),
        _rep(lm_w2t), _rep(lm_b2),
    ]
    out = pl.pallas_call(
        functools.partial(_sim_kernel, nb, N),
        out_shape=jax.ShapeDtypeStruct((1, B * N), jnp.float32),
        grid=(B // nb,),
        in_specs=in_specs,
        out_specs=pl.BlockSpec((1, nb * N), lambda i: (0, i)),
        compiler_params=pltpu.CompilerParams(
            dimension_semantics=("parallel",),
            vmem_limit_bytes=64 * 1024 * 1024,
        ),
    )(xt, tb3, se_w0, se_b0, se_w1, se_b1, ws,
      lw1t, lm_b1, lm_w2t, lm_b2)

    return out.reshape(B, N, 1)


# speedup vs baseline: 5.4654x; 1.0131x over previous
import functools

import jax
import jax.numpy as jnp
from jax import lax
from jax.experimental import pallas as pl
from jax.experimental.pallas import tpu as pltpu


def _theta_kernel(theta_t_ref, tw0, tb0, tw1, tb1, tw2_t, tb2,
                  sw2_t, sb2, wl0, bl0, tb_out, ws_out):
    ta = (((0,), (0,)), ((), ()))
    t = lax.dot_general(theta_t_ref[...], tw0[...], ta,
                        preferred_element_type=jnp.float32)
    t = jnp.maximum(t + tb0[...], 0.0)
    t = jnp.maximum(jnp.dot(t, tw1[...], preferred_element_type=jnp.float32)
                    + tb1[...], 0.0)
    wl0_t = wl0[0:32, :]
    wl0_s = wl0[32:64, :]
    w_t = lax.dot_general(tw2_t[...], wl0_t, ta,
                          preferred_element_type=jnp.float32)
    b_t = (jnp.dot(tb2[...], wl0_t, preferred_element_type=jnp.float32)
           + jnp.dot(sb2[...], wl0_s, preferred_element_type=jnp.float32)
           + bl0[...])
    tb_out[...] = (jnp.dot(t, w_t, preferred_element_type=jnp.float32) + b_t)
    ws_out[...] = lax.dot_general(sw2_t[...], wl0_s, ta,
                                  preferred_element_type=jnp.float32)


def _sim_kernel(nb, n_tgt, x_ref, tb_ref, w0, b0, w1, b1, ws,
                lw1_t, lb1, lw2t, lb2, o_ref):
    w0b = w0[...].astype(jnp.bfloat16)
    parts = []
    for i in range(nb):
        xi = x_ref[i].astype(jnp.bfloat16)
        parts.append(lax.dot_general(xi, w0b, (((0,), (0,)), ((), ())),
                                     preferred_element_type=jnp.float32))
    h = jnp.concatenate(parts, axis=0)
    h = jnp.maximum(h + b0[...], 0.0).astype(jnp.bfloat16)
    h = jnp.dot(h, w1[...].astype(jnp.bfloat16),
                preferred_element_type=jnp.float32)
    h = jnp.maximum(h + b1[...], 0.0).astype(jnp.bfloat16)
    h = jnp.dot(h, ws[...].astype(jnp.bfloat16),
                preferred_element_type=jnp.float32)
    h = h.reshape(nb, n_tgt, h.shape[-1]) + tb_ref[...]
    h = jnp.maximum(h, 0.0).reshape(nb * n_tgt, h.shape[-1]).astype(jnp.bfloat16)
    h = lax.dot_general(h, lw1_t[...].astype(jnp.bfloat16),
                        (((1,), (1,)), ((), ())),
                        preferred_element_type=jnp.float32)
    h = jnp.maximum(h + lb1[...], 0.0)
    row = lax.dot_general(lw2t[...], h,
                          (((1,), (1,)), ((), ())),
                          preferred_element_type=jnp.float32) + lb2[...]
    o_ref[...] = row


def _rep(arr):
    zeros = (0,) * arr.ndim
    return pl.BlockSpec(arr.shape, lambda *_: zeros)


def kernel(theta, x_target, te_w0, te_b0, te_w1, te_b1, te_w2, te_b2,
           se_w0, se_b0, se_w1, se_b1, se_w2, se_b2,
           lm_w0, lm_b0, lm_w1, lm_b1, lm_w2, lm_b2):
    B, theta_dim = theta.shape
    _, N, sim_dim = x_target.shape
    h0 = lm_w0.shape[1]

    tb, ws = pl.pallas_call(
        _theta_kernel,
        out_shape=[
            jax.ShapeDtypeStruct((B, h0), jnp.float32),
            jax.ShapeDtypeStruct((se_w2.shape[0], h0), jnp.float32),
        ],
        compiler_params=pltpu.CompilerParams(
            vmem_limit_bytes=64 * 1024 * 1024,
        ),
    )(jnp.swapaxes(theta, 0, 1), te_w0, te_b0, te_w1, te_b1,
      jnp.swapaxes(te_w2, 0, 1), te_b2,
      jnp.swapaxes(se_w2, 0, 1), se_b2, lm_w0, lm_b0)

    nb = 16
    while B % nb:
        nb //= 2
    xt = jnp.swapaxes(x_target, 1, 2)
    lw1t = jnp.swapaxes(lm_w1, 0, 1)
    lm_w2t = lm_w2.reshape(1, lm_w2.shape[0])
    tb3 = tb.reshape(B, 1, h0)

    in_specs = [
        pl.BlockSpec((nb, sim_dim, N), lambda i: (i, 0, 0)),
        pl.BlockSpec((nb, 1, h0), lambda i: (i, 0, 0)),
        _rep(se_w0), _rep(se_b0), _rep(se_w1), _rep(se_b1),
        _rep(ws), _rep(lw1t), _rep(lm_b1),
        _rep(lm_w2t), _rep(lm_b2),
    ]
    out = pl.pallas_call(
        functools.partial(_sim_kernel, nb, N),
        out_shape=jax.ShapeDtypeStruct((1, B * N), jnp.float32),
        grid=(B // nb,),
        in_specs=in_specs,
        out_specs=pl.BlockSpec((1, nb * N), lambda i: (0, i)),
        compiler_params=pltpu.CompilerParams(
            dimension_semantics=("parallel",),
            vmem_limit_bytes=64 * 1024 * 1024,
        ),
    )(xt, tb3, se_w0, se_b0, se_w1, se_b1, ws,
      lw1t, lm_b1, lm_w2t, lm_b2)

    return out.reshape(B, N, 1)


# speedup vs baseline: 5.4691x; 1.0007x over previous
import functools

import jax
import jax.numpy as jnp
from jax import lax
from jax.experimental import pallas as pl
from jax.experimental.pallas import tpu as pltpu


def _theta_kernel(theta_t_ref, tw0, tb0, tw1, tb1, tw2_t, tb2,
                  sw2_t, sb2, wl0, bl0, tb_out, ws_out):
    ta = (((0,), (0,)), ((), ()))
    t = lax.dot_general(theta_t_ref[...], tw0[...], ta,
                        preferred_element_type=jnp.float32)
    t = jnp.maximum(t + tb0[...], 0.0)
    t = jnp.maximum(jnp.dot(t, tw1[...], preferred_element_type=jnp.float32)
                    + tb1[...], 0.0)
    wl0_t = wl0[0:32, :]
    wl0_s = wl0[32:64, :]
    w_t = lax.dot_general(tw2_t[...], wl0_t, ta,
                          preferred_element_type=jnp.float32)
    b_t = (jnp.dot(tb2[...], wl0_t, preferred_element_type=jnp.float32)
           + jnp.dot(sb2[...], wl0_s, preferred_element_type=jnp.float32)
           + bl0[...])
    tb_out[...] = (jnp.dot(t, w_t, preferred_element_type=jnp.float32) + b_t)
    ws_out[...] = lax.dot_general(sw2_t[...], wl0_s, ta,
                                  preferred_element_type=jnp.float32)


def _sim_kernel(nb, n_tgt, x_ref, tb_ref, w0, b0, w1, b1, ws,
                lw1_t, lb1, lw2t, lb2, o_ref):
    w0b = w0[...].astype(jnp.bfloat16)
    parts = []
    for i in range(nb):
        xi = x_ref[i].astype(jnp.bfloat16)
        parts.append(lax.dot_general(xi, w0b, (((0,), (0,)), ((), ())),
                                     preferred_element_type=jnp.float32))
    h = jnp.concatenate(parts, axis=0)
    h = jnp.maximum(h + b0[...], 0.0).astype(jnp.bfloat16)
    h = jnp.dot(h, w1[...].astype(jnp.bfloat16),
                preferred_element_type=jnp.float32)
    h = jnp.maximum(h + b1[...], 0.0).astype(jnp.bfloat16)
    h = jnp.dot(h, ws[...].astype(jnp.bfloat16),
                preferred_element_type=jnp.float32)
    h = h.reshape(nb, n_tgt, h.shape[-1]) + tb_ref[...]
    h = jnp.maximum(h, 0.0).reshape(nb * n_tgt, h.shape[-1]).astype(jnp.bfloat16)
    h = lax.dot_general(h, lw1_t[...].astype(jnp.bfloat16),
                        (((1,), (1,)), ((), ())),
                        preferred_element_type=jnp.float32)
    h = jnp.maximum(h + lb1[...], 0.0)
    row = lax.dot_general(lw2t[...], h,
                          (((1,), (1,)), ((), ())),
                          preferred_element_type=jnp.float32) + lb2[...]
    o_ref[...] = row


def _rep(arr):
    zeros = (0,) * arr.ndim
    return pl.BlockSpec(arr.shape, lambda *_: zeros)


def kernel(theta, x_target, te_w0, te_b0, te_w1, te_b1, te_w2, te_b2,
           se_w0, se_b0, se_w1, se_b1, se_w2, se_b2,
           lm_w0, lm_b0, lm_w1, lm_b1, lm_w2, lm_b2):
    B, theta_dim = theta.shape
    _, N, sim_dim = x_target.shape
    h0 = lm_w0.shape[1]

    tb, ws = pl.pallas_call(
        _theta_kernel,
        out_shape=[
            jax.ShapeDtypeStruct((B, h0), jnp.float32),
            jax.ShapeDtypeStruct((se_w2.shape[0], h0), jnp.float32),
        ],
        compiler_params=pltpu.CompilerParams(
            vmem_limit_bytes=64 * 1024 * 1024,
        ),
    )(jnp.swapaxes(theta, 0, 1), te_w0, te_b0, te_w1, te_b1,
      jnp.swapaxes(te_w2, 0, 1), te_b2,
      jnp.swapaxes(se_w2, 0, 1), se_b2, lm_w0, lm_b0)

    nb = 16
    while B % nb:
        nb //= 2
    xt = jnp.swapaxes(x_target, 1, 2)
    lw1t = jnp.swapaxes(lm_w1, 0, 1)
    lm_w2t = lm_w2.reshape(1, lm_w2.shape[0])
    tb3 = tb.reshape(B, 1, h0)

    in_specs = [
        pl.BlockSpec((nb, sim_dim, N), lambda i: (i, 0, 0)),
        pl.BlockSpec((nb, 1, h0), lambda i: (i, 0, 0)),
        _rep(se_w0), _rep(se_b0), _rep(se_w1), _rep(se_b1),
        _rep(ws), _rep(lw1t), _rep(lm_b1),
        _rep(lm_w2t), _rep(lm_b2),
    ]
    out = pl.pallas_call(
        functools.partial(_sim_kernel, nb, N),
        out_shape=jax.ShapeDtypeStruct((1, B * N), jnp.float32),
        grid=(B // nb,),
        in_specs=in_specs,
        out_specs=pl.BlockSpec((1, nb * N), lambda i: (0, i)),
        compiler_params=pltpu.CompilerParams(
            dimension_semantics=("parallel",),
            vmem_limit_bytes=64 * 1024 * 1024,
            fuse_transposed_lhs_in_matmul=True,
        ),
    )(xt, tb3, se_w0, se_b0, se_w1, se_b1, ws,
      lw1t, lm_b1, lm_w2t, lm_b2)

    return out.reshape(B, N, 1)


# speedup vs baseline: 5.5235x; 1.0099x over previous
import functools

import jax
import jax.numpy as jnp
from jax import lax
from jax.experimental import pallas as pl
from jax.experimental.pallas import tpu as pltpu


def _theta_kernel(theta_t_ref, tw0, tb0, tw1, tb1, tw2_t, tb2,
                  sw2_t, sb2, wl0, bl0, tb_out, ws_out):
    ta = (((0,), (0,)), ((), ()))
    t = lax.dot_general(theta_t_ref[...], tw0[...], ta,
                        preferred_element_type=jnp.float32)
    t = jnp.maximum(t + tb0[...], 0.0)
    t = jnp.maximum(jnp.dot(t, tw1[...], preferred_element_type=jnp.float32)
                    + tb1[...], 0.0)
    wl0_t = wl0[0:32, :]
    wl0_s = wl0[32:64, :]
    w_t = lax.dot_general(tw2_t[...], wl0_t, ta,
                          preferred_element_type=jnp.float32)
    b_t = (jnp.dot(tb2[...], wl0_t, preferred_element_type=jnp.float32)
           + jnp.dot(sb2[...], wl0_s, preferred_element_type=jnp.float32)
           + bl0[...])
    tb_out[...] = (jnp.dot(t, w_t, preferred_element_type=jnp.float32) + b_t)
    ws_out[...] = lax.dot_general(sw2_t[...], wl0_s, ta,
                                  preferred_element_type=jnp.float32)


def _sim_kernel(nb, n_tgt, x_ref, tb_ref, w0, b0, w1, b1, ws,
                lw1_t, lb1, lw2t, lb2, o_ref):
    w0b = w0[...].astype(jnp.bfloat16)
    parts = []
    for i in range(nb):
        xi = x_ref[i].astype(jnp.bfloat16)
        parts.append(lax.dot_general(xi, w0b, (((0,), (0,)), ((), ())),
                                     preferred_element_type=jnp.float32))
    h = jnp.concatenate(parts, axis=0)
    h = jnp.maximum(h + b0[...], 0.0).astype(jnp.bfloat16)
    h = jnp.dot(h, w1[...].astype(jnp.bfloat16),
                preferred_element_type=jnp.float32)
    h = jnp.maximum(h + b1[...], 0.0).astype(jnp.bfloat16)
    h = jnp.dot(h, ws[...].astype(jnp.bfloat16),
                preferred_element_type=jnp.float32)
    h = h.reshape(nb, n_tgt, h.shape[-1]) + tb_ref[...]
    h = jnp.maximum(h, 0.0).reshape(nb * n_tgt, h.shape[-1]).astype(jnp.bfloat16)
    h = lax.dot_general(h, lw1_t[...].astype(jnp.bfloat16),
                        (((1,), (1,)), ((), ())),
                        preferred_element_type=jnp.float32)
    h = jnp.maximum(h + lb1[...], 0.0)
    row = lax.dot_general(lw2t[...], h,
                          (((1,), (1,)), ((), ())),
                          preferred_element_type=jnp.float32) + lb2[...]
    o_ref[...] = row


def _rep(arr):
    zeros = (0,) * arr.ndim
    return pl.BlockSpec(arr.shape, lambda *_: zeros)


def kernel(theta, x_target, te_w0, te_b0, te_w1, te_b1, te_w2, te_b2,
           se_w0, se_b0, se_w1, se_b1, se_w2, se_b2,
           lm_w0, lm_b0, lm_w1, lm_b1, lm_w2, lm_b2):
    B, theta_dim = theta.shape
    _, N, sim_dim = x_target.shape
    h0 = lm_w0.shape[1]

    tb, ws = pl.pallas_call(
        _theta_kernel,
        out_shape=[
            jax.ShapeDtypeStruct((B, h0), jnp.float32),
            jax.ShapeDtypeStruct((se_w2.shape[0], h0), jnp.float32),
        ],
        compiler_params=pltpu.CompilerParams(
            vmem_limit_bytes=64 * 1024 * 1024,
        ),
    )(jnp.swapaxes(theta, 0, 1), te_w0, te_b0, te_w1, te_b1,
      jnp.swapaxes(te_w2, 0, 1), te_b2,
      jnp.swapaxes(se_w2, 0, 1), se_b2, lm_w0, lm_b0)

    nb = 32
    while B % nb:
        nb //= 2
    xt = jnp.swapaxes(x_target, 1, 2)
    lw1t = jnp.swapaxes(lm_w1, 0, 1)
    lm_w2t = lm_w2.reshape(1, lm_w2.shape[0])
    tb3 = tb.reshape(B, 1, h0)

    in_specs = [
        pl.BlockSpec((nb, sim_dim, N), lambda i: (i, 0, 0)),
        pl.BlockSpec((nb, 1, h0), lambda i: (i, 0, 0)),
        _rep(se_w0), _rep(se_b0), _rep(se_w1), _rep(se_b1),
        _rep(ws), _rep(lw1t), _rep(lm_b1),
        _rep(lm_w2t), _rep(lm_b2),
    ]
    out = pl.pallas_call(
        functools.partial(_sim_kernel, nb, N),
        out_shape=jax.ShapeDtypeStruct((1, B * N), jnp.float32),
        grid=(B // nb,),
        in_specs=in_specs,
        out_specs=pl.BlockSpec((1, nb * N), lambda i: (0, i)),
        compiler_params=pltpu.CompilerParams(
            dimension_semantics=("parallel",),
            vmem_limit_bytes=64 * 1024 * 1024,
            fuse_transposed_lhs_in_matmul=True,
        ),
    )(xt, tb3, se_w0, se_b0, se_w1, se_b1, ws,
      lw1t, lm_b1, lm_w2t, lm_b2)

    return out.reshape(B, N, 1)
